# Initial kernel scaffold; baseline (speedup 1.0000x reference)
import jax
import jax.numpy as jnp
from jax.experimental import pallas as pl


def kernel(x, table):
    raise NotImplementedError("write your pallas kernel here")


# speedup vs baseline: 1.6293x; 1.6293x over previous
import functools

import jax
import jax.numpy as jnp
from jax import lax
from jax.experimental import pallas as pl
from jax.experimental.pallas import tpu as pltpu
from jax.experimental.pallas import tpu_sc as plsc

_VOCAB = 100000
_D = 1024
_B_TOT = 4 * 8192
_NC = 2
_NS = 16
_NW = _NC * _NS
_B_PER_W = _B_TOT // _NW
_CHUNK = 64
_N_CHUNKS = _B_PER_W // _CHUNK

_mesh = plsc.VectorSubcoreMesh(
    core_axis_name="c", subcore_axis_name="s", num_cores=_NC, num_subcores=_NS
)


@functools.partial(
    pl.kernel,
    out_type=jax.ShapeDtypeStruct((_B_TOT, _D), jnp.float32),
    mesh=_mesh,
    scratch_types=[
        pltpu.VMEM((_B_PER_W,), jnp.int32),
        pltpu.VMEM((_CHUNK, _D), jnp.float32),
        pltpu.SemaphoreType.DMA,
    ],
)
def _gather_rows(idx_hbm, table_hbm, out_hbm, idx_v, rows_v, sem):
    wid = lax.axis_index("s") * _NC + lax.axis_index("c")
    base = wid * _B_PER_W
    pltpu.sync_copy(idx_hbm.at[pl.ds(base, _B_PER_W)], idx_v)

    def body(g, _):
        off = g * _CHUNK
        pltpu.async_copy(
            table_hbm.at[idx_v.at[pl.ds(off, _CHUNK)]], rows_v, sem
        ).wait()
        pltpu.sync_copy(rows_v, out_hbm.at[pl.ds(base + off, _CHUNK)])
        return 0

    lax.fori_loop(0, _N_CHUNKS, body, 0)


def kernel(x, table):
    idx = x.reshape(-1).astype(jnp.int32)
    out = _gather_rows(idx, table)
    return out.reshape(x.shape + (table.shape[1],))


# speedup vs baseline: 1.7763x; 1.0902x over previous
import functools

import jax
import jax.numpy as jnp
from jax import lax
from jax.experimental import pallas as pl
from jax.experimental.pallas import tpu as pltpu
from jax.experimental.pallas import tpu_sc as plsc

_VOCAB = 100000
_D = 1024
_B_TOT = 4 * 8192
_NC = 2
_NS = 16
_NW = _NC * _NS
_B_PER_W = _B_TOT // _NW
_CHUNK = 32
_N_CHUNKS = _B_PER_W // _CHUNK
_NBUF = 2

_mesh = plsc.VectorSubcoreMesh(
    core_axis_name="c", subcore_axis_name="s", num_cores=_NC, num_subcores=_NS
)


@functools.partial(
    pl.kernel,
    out_type=jax.ShapeDtypeStruct((_B_TOT, _D), jnp.float32),
    mesh=_mesh,
    scratch_types=[
        pltpu.VMEM((_B_PER_W,), jnp.int32),
        pltpu.VMEM((_NBUF, _CHUNK, _D), jnp.float32),
        [pltpu.SemaphoreType.DMA] * _NBUF,
        [pltpu.SemaphoreType.DMA] * _NBUF,
    ],
)
def _gather_rows(idx_hbm, table_hbm, out_hbm, idx_v, rows_v, gsems, osems):
    wid = lax.axis_index("s") * _NC + lax.axis_index("c")
    base = wid * _B_PER_W
    pltpu.sync_copy(idx_hbm.at[pl.ds(base, _B_PER_W)], idx_v)

    def start_gather(g, slot):
        pltpu.async_copy(
            table_hbm.at[idx_v.at[pl.ds(g * _CHUNK, _CHUNK)]],
            rows_v.at[slot],
            gsems[slot],
        )

    def wait_gather(slot):
        pltpu.make_async_copy(
            table_hbm.at[idx_v.at[pl.ds(0, _CHUNK)]], rows_v.at[slot], gsems[slot]
        ).wait()

    def start_scatter(g, slot):
        pltpu.async_copy(
            rows_v.at[slot], out_hbm.at[pl.ds(base + g * _CHUNK, _CHUNK)], osems[slot]
        )

    def wait_scatter(slot):
        pltpu.make_async_copy(
            rows_v.at[slot], out_hbm.at[pl.ds(base, _CHUNK)], osems[slot]
        ).wait()

    for s in range(_NBUF):
        start_gather(s, s)

    def body(o, _):
        for s in range(_NBUF):
            g = o * _NBUF + s
            wait_gather(s)
            start_scatter(g, s)

            @pl.when(g + _NBUF < _N_CHUNKS)
            def _():
                wait_scatter(s)
                start_gather(g + _NBUF, s)

        return 0

    lax.fori_loop(0, _N_CHUNKS // _NBUF, body, 0)
    for s in range(_NBUF):
        wait_scatter(s)


def kernel(x, table):
    idx = x.reshape(-1).astype(jnp.int32)
    out = _gather_rows(idx, table)
    return out.reshape(x.shape + (table.shape[1],))


# speedup vs baseline: 1.7786x; 1.0013x over previous
import functools

import jax
import jax.numpy as jnp
from jax import lax
from jax.experimental import pallas as pl
from jax.experimental.pallas import tpu as pltpu
from jax.experimental.pallas import tpu_sc as plsc

_VOCAB = 100000
_D = 1024
_B_TOT = 4 * 8192
_NC = 2
_NS = 16
_NW = _NC * _NS
_B_PER_W = _B_TOT // _NW
_CHUNK = 32
_N_CHUNKS = _B_PER_W // _CHUNK
_NBUF = 3

_mesh = plsc.VectorSubcoreMesh(
    core_axis_name="c", subcore_axis_name="s", num_cores=_NC, num_subcores=_NS
)


@functools.partial(
    pl.kernel,
    out_type=jax.ShapeDtypeStruct((_B_TOT, _D), jnp.float32),
    mesh=_mesh,
    scratch_types=[
        pltpu.VMEM((_B_PER_W,), jnp.int32),
        pltpu.VMEM((_NBUF, _CHUNK, _D), jnp.float32),
        [pltpu.SemaphoreType.DMA] * _NBUF,
        [pltpu.SemaphoreType.DMA] * _NBUF,
    ],
)
def _gather_rows(idx_hbm, table_hbm, out_hbm, idx_v, rows_v, gsems, osems):
    wid = lax.axis_index("s") * _NC + lax.axis_index("c")
    base = wid * _B_PER_W
    pltpu.sync_copy(idx_hbm.at[pl.ds(base, _B_PER_W)], idx_v)

    def start_gather(g, slot):
        pltpu.async_copy(
            table_hbm.at[idx_v.at[pl.ds(g * _CHUNK, _CHUNK)]],
            rows_v.at[slot],
            gsems[slot],
        )

    def wait_gather(slot):
        pltpu.make_async_copy(
            table_hbm.at[idx_v.at[pl.ds(0, _CHUNK)]], rows_v.at[slot], gsems[slot]
        ).wait()

    def start_scatter(g, slot):
        pltpu.async_copy(
            rows_v.at[slot], out_hbm.at[pl.ds(base + g * _CHUNK, _CHUNK)], osems[slot]
        )

    def wait_scatter(slot):
        pltpu.make_async_copy(
            rows_v.at[slot], out_hbm.at[pl.ds(base, _CHUNK)], osems[slot]
        ).wait()

    for s in range(min(_NBUF, _N_CHUNKS)):
        start_gather(s, s)

    def body(o, _):
        for s in range(_NBUF):
            g = o * _NBUF + s
            wait_gather(s)
            start_scatter(g, s)

            @pl.when(g + _NBUF < _N_CHUNKS)
            def _():
                wait_scatter(s)
                start_gather(g + _NBUF, s)

        return 0

    lax.fori_loop(0, _N_CHUNKS // _NBUF, body, 0)
    for g in range(_N_CHUNKS - _N_CHUNKS % _NBUF, _N_CHUNKS):
        s = g % _NBUF
        wait_gather(s)
        start_scatter(g, s)
    for s in range(min(_NBUF, _N_CHUNKS)):
        wait_scatter(s)


def kernel(x, table):
    idx = x.reshape(-1).astype(jnp.int32)
    out = _gather_rows(idx, table)
    return out.reshape(x.shape + (table.shape[1],))


# speedup vs baseline: 1.7823x; 1.0021x over previous
import functools

import jax
import jax.numpy as jnp
from jax import lax
from jax.experimental import pallas as pl
from jax.experimental.pallas import tpu as pltpu
from jax.experimental.pallas import tpu_sc as plsc

_VOCAB = 100000
_D = 1024
_B_TOT = 4 * 8192
_NC = 2
_NS = 16
_NW = _NC * _NS
_B_PER_W = _B_TOT // _NW
_CHUNK = 32
_N_CHUNKS = _B_PER_W // _CHUNK
_NBUF = 3

_mesh = plsc.VectorSubcoreMesh(
    core_axis_name="c", subcore_axis_name="s", num_cores=_NC, num_subcores=_NS
)


_BATCH = 4
_SEQ = 8192
_W_PER_BATCH = _SEQ // _B_PER_W


@functools.partial(
    pl.kernel,
    out_type=jax.ShapeDtypeStruct((_BATCH, _SEQ, _D), jnp.float32),
    mesh=_mesh,
    scratch_types=[
        pltpu.VMEM((_B_PER_W,), jnp.int32),
        pltpu.VMEM((_NBUF, _CHUNK, _D), jnp.float32),
        [pltpu.SemaphoreType.DMA] * _NBUF,
        [pltpu.SemaphoreType.DMA] * _NBUF,
    ],
)
def _gather_rows(idx_hbm, table_hbm, out_hbm, idx_v, rows_v, gsems, osems):
    wid = lax.axis_index("s") * _NC + lax.axis_index("c")
    b = wid // _W_PER_BATCH
    base = (wid % _W_PER_BATCH) * _B_PER_W
    pltpu.sync_copy(idx_hbm.at[b, pl.ds(base, _B_PER_W)], idx_v)

    def start_gather(g, slot):
        pltpu.async_copy(
            table_hbm.at[idx_v.at[pl.ds(g * _CHUNK, _CHUNK)]],
            rows_v.at[slot],
            gsems[slot],
        )

    def wait_gather(slot):
        pltpu.make_async_copy(
            table_hbm.at[idx_v.at[pl.ds(0, _CHUNK)]], rows_v.at[slot], gsems[slot]
        ).wait()

    def start_scatter(g, slot):
        pltpu.async_copy(
            rows_v.at[slot],
            out_hbm.at[b, pl.ds(base + g * _CHUNK, _CHUNK)],
            osems[slot],
        )

    def wait_scatter(slot):
        pltpu.make_async_copy(
            rows_v.at[slot], out_hbm.at[b, pl.ds(base, _CHUNK)], osems[slot]
        ).wait()

    for s in range(min(_NBUF, _N_CHUNKS)):
        start_gather(s, s)

    def body(o, _):
        for s in range(_NBUF):
            g = o * _NBUF + s
            wait_gather(s)
            start_scatter(g, s)

            @pl.when(g + _NBUF < _N_CHUNKS)
            def _():
                wait_scatter(s)
                start_gather(g + _NBUF, s)

        return 0

    lax.fori_loop(0, _N_CHUNKS // _NBUF, body, 0)
    for g in range(_N_CHUNKS - _N_CHUNKS % _NBUF, _N_CHUNKS):
        s = g % _NBUF
        wait_gather(s)
        start_scatter(g, s)
    for s in range(min(_NBUF, _N_CHUNKS)):
        wait_scatter(s)


def kernel(x, table):
    if x.dtype != jnp.int32:
        x = x.astype(jnp.int32)
    return _gather_rows(x, table)
